# baseline (speedup 1.0000x reference)
import jax
import jax.numpy as jnp
from jax import lax
from jax.experimental import pallas as pl
from jax.experimental.pallas import tpu as pltpu
from jax.experimental.pallas import tpu_sc as plsc

SIZE = 1383
P = SIZE * SIZE
N_SIG = 1024 * 1024
NT = 11
ROWW = NT * 256
WPHYS = SIZE * ROWW

SSTR = SIZE + 1
H1_ROWS = 696
H2_ROWS = SIZE - H1_ROWS
H1_SP = H1_ROWS * SSTR
H2_SP = H2_ROWS * SSTR
DUMP1 = H1_SP
DUMP2 = H2_SP
DSPREAD = 2048
SP_ALLOC = H1_SP + DSPREAD

RB = 8
A_LEN = RB * SIZE
C_LEN = RB * SSTR
B_LEN = RB * ROWW
N1BLK = H1_ROWS // RB
N2BLK = H2_ROWS // RB
TAIL_R = H2_ROWS - N2BLK * RB

NZF = SP_ALLOC // B_LEN
ZTAIL = SP_ALLOC - NZF * B_LEN

NW = 16
ELEMS_PER_W = N_SIG // NW
SUB = 16
SCHUNK = ELEMS_PER_W // SUB


def _body(phi2_hbm, val_hbm, idxa_hbm, idxb_hbm, out_hbm,
          a_ref, c_ref, b_ref, idx0, idx1, val0, val1,
          raster, lA, lB, sA, sB):
    wid = lax.axis_index("s")
    iota16 = lax.iota(jnp.int32, 16)
    zeros16 = jnp.zeros((16,), jnp.float32)
    idx_bufs, val_bufs = (idx0, idx1), (val0, val1)
    lsems, ssems = (lA, lB), (sA, sB)

    def zbody(i, c):
        b_ref[pl.ds(i * 16, 16)] = zeros16
        return c
    lax.fori_loop(0, B_LEN // 16, zbody, 0, unroll=8)

    def zero_raster():
        def zc(k, c):
            ch = wid + k * NW
            @pl.when(ch < NZF)
            def _():
                pltpu.sync_copy(b_ref, raster.at[pl.ds(ch * B_LEN, B_LEN)])
            return c
        lax.fori_loop(0, (NZF + NW - 1) // NW, zc, 0)
        @pl.when(wid == 11)
        def _():
            pltpu.sync_copy(b_ref.at[pl.ds(0, ZTAIL)],
                            raster.at[pl.ds(NZF * B_LEN, ZTAIL)])

    def scatter_pass(idx_hbm):
        base = wid * ELEMS_PER_W

        def ld_descs(s, buf):
            e0 = base + s * SCHUNK
            return (pltpu.make_async_copy(idx_hbm.at[pl.ds(e0, SCHUNK)],
                                          idx_bufs[buf], lsems[buf]),
                    pltpu.make_async_copy(val_hbm.at[pl.ds(e0, SCHUNK)],
                                          val_bufs[buf], lsems[buf]))

        def sc_desc(buf):
            return pltpu.make_async_copy(val_bufs[buf],
                                         raster.at[idx_bufs[buf]], ssems[buf])

        def sc_slot(s, t):
            @pl.when(jnp.logical_and(s >= 1, s + 1 < SUB))
            def _():
                sc_desc(1 - t).wait()
            @pl.when(s + 1 < SUB)
            def _():
                for de in ld_descs(s + 1, 1 - t):
                    de.start()
            @pl.when(s < SUB)
            def _():
                for de in ld_descs(s, t):
                    de.wait()
                sc_desc(t).start()

        for de in ld_descs(0, 0):
            de.start()
        def sc_pair(q, c):
            sc_slot(2 * q, 0)
            sc_slot(2 * q + 1, 1)
            return c
        lax.fori_loop(0, SUB // 2, sc_pair, 0)
        sc_desc((SUB - 2) % 2).wait()
        sc_desc((SUB - 1) % 2).wait()

    def copy_rows(nrows):
        def row(r, c2):
            ssrc = r * SSTR
            bdst = r * ROWW
            def tile(jt, c3):
                for k in range(8):
                    sv = plsc.load_gather(
                        c_ref, [ssrc + jt * 128 + k * 16 + iota16])
                    b_ref[pl.ds(bdst + jt * 256 + k * 16, 16)] = sv
                    pv = a_ref[jt, r, pl.ds(k * 16, 16)]
                    b_ref[pl.ds(bdst + jt * 256 + 128 + k * 16, 16)] = pv
                return c3
            lax.fori_loop(0, NT - 1, tile, 0)
            for k in range(6):
                sv = plsc.load_gather(c_ref, [ssrc + 1280 + k * 16 + iota16])
                b_ref[pl.ds(bdst + 2560 + k * 16, 16)] = sv
                pv = a_ref[10, r, pl.ds(k * 16, 16)]
                b_ref[pl.ds(bdst + 2688 + k * 16, 16)] = pv
            m7 = iota16 < 7
            sv = plsc.load_gather(c_ref, [ssrc + 1376 + iota16])
            plsc.store_scatter(b_ref, [bdst + 2656 + iota16], sv, mask=m7)
            pv = a_ref[10, r, pl.ds(96, 16)]
            plsc.store_scatter(b_ref, [bdst + 2784 + iota16], pv, mask=m7)
            return c2
        lax.fori_loop(0, nrows, row, 0)

    def merge_pass(nblk, row_base):
        def mslot(k, c):
            blk = wid + k * NW
            @pl.when(blk < nblk)
            def _():
                g0 = row_base + blk * RB
                pltpu.sync_copy(phi2_hbm.at[row_base // RB + blk], a_ref)
                pltpu.sync_copy(
                    raster.at[pl.ds(blk * RB * SSTR, C_LEN)],
                    c_ref.at[pl.ds(0, C_LEN)])
                copy_rows(RB)
                pltpu.sync_copy(
                    b_ref, out_hbm.at[pl.ds(g0 * ROWW, B_LEN)])
            return c
        lax.fori_loop(0, (nblk + NW - 1) // NW, mslot, 0)

    zero_raster()
    plsc.subcore_barrier()
    scatter_pass(idxa_hbm)
    plsc.subcore_barrier()
    merge_pass(N1BLK, 0)
    plsc.subcore_barrier()

    lax.fori_loop(0, B_LEN // 16, zbody, 0, unroll=8)
    zero_raster()
    plsc.subcore_barrier()
    scatter_pass(idxb_hbm)
    plsc.subcore_barrier()
    merge_pass(N2BLK, H1_ROWS)

    @pl.when(wid == NW - 1)
    def _():
        pltpu.sync_copy(phi2_hbm.at[172], a_ref)
        pltpu.sync_copy(raster.at[pl.ds(N2BLK * RB * SSTR, TAIL_R * SSTR)],
                        c_ref.at[pl.ds(0, TAIL_R * SSTR)])
        copy_rows(TAIL_R)
        pltpu.sync_copy(b_ref.at[pl.ds(0, TAIL_R * ROWW)],
                        out_hbm.at[pl.ds(1376 * ROWW, TAIL_R * ROWW)])


def kernel(x, phi2, koordinates):
    phi2t = jnp.pad(phi2, ((0, 1), (0, 25))).reshape(173, 8, 11, 128)
    phi2t = phi2t.transpose(0, 2, 1, 3)
    xf = x.reshape(-1)
    p = koordinates[:N_SIG, 0].astype(jnp.int32)
    row = p // SIZE
    col = p - row * SIZE
    spread = jnp.bitwise_and(p, DSPREAD - 1)
    idxa = jnp.where(row < H1_ROWS, row * SSTR + col, DUMP1 + spread)
    idxb = jnp.where(row >= H1_ROWS, (row - H1_ROWS) * SSTR + col,
                     DUMP2 + spread)

    mesh = plsc.VectorSubcoreMesh(core_axis_name="c", subcore_axis_name="s",
                                  num_cores=1, num_subcores=NW)
    out = pl.kernel(
        _body,
        out_type=jax.ShapeDtypeStruct((WPHYS,), jnp.float32),
        mesh=mesh,
        compiler_params=pltpu.CompilerParams(needs_layout_passes=False),
        scratch_types=[
            pltpu.VMEM((11, 8, 128), jnp.float32),
            pltpu.VMEM((C_LEN + 16,), jnp.float32),
            pltpu.VMEM((B_LEN,), jnp.float32),
            pltpu.VMEM((SCHUNK,), jnp.int32),
            pltpu.VMEM((SCHUNK,), jnp.int32),
            pltpu.VMEM((SCHUNK,), jnp.float32),
            pltpu.VMEM((SCHUNK,), jnp.float32),
            pltpu.VMEM_SHARED((SP_ALLOC,), jnp.float32),
            pltpu.SemaphoreType.DMA,
            pltpu.SemaphoreType.DMA,
            pltpu.SemaphoreType.DMA,
            pltpu.SemaphoreType.DMA,
        ],
    )(phi2t, xf, idxa, idxb)
    a4 = out.reshape(SIZE, NT, 2, 128)
    full = a4.transpose(0, 1, 3, 2).reshape(SIZE, NT * 128, 2)
    return full[:, :SIZE, :].reshape(1, SIZE, SIZE, 2)
